# Initial kernel scaffold; baseline (speedup 1.0000x reference)
import jax
import jax.numpy as jnp
from jax.experimental import pallas as pl


def kernel(x, edge_index, Wl0, bl0, Wr0, br0, att0, b0, g0, be0, Wl1, bl1, Wr1, br1, att1, b1, g1, be1, W1, bv1, W2, bv2):
    raise NotImplementedError("write your pallas kernel here")


# speedup vs baseline: 7.3376x; 7.3376x over previous
import functools

import jax
import jax.numpy as jnp
from jax import lax
from jax.experimental import pallas as pl
from jax.experimental.pallas import tpu as pltpu
from jax.experimental.pallas import tpu_sc as plsc

N = 10000
E = 320000
C = 128
OUT = 64
NPAD = 10000
NCORE = 2
NSUB = 16
NWORK = NCORE * NSUB
EPW = E // NWORK
B = 80
NB = EPW // B
NCHUNK = NPAD // B
_NCHUNK_CEIL = -(-NCHUNK // NSUB)


_GDN = lax.GatherDimensionNumbers(
    offset_dims=(), collapsed_slice_dims=(0,), start_index_map=(0,))


def _lane_shuffle(v, idx):
    return lax.gather(v, idx[:, None], _GDN, slice_sizes=(1,),
                      mode=lax.GatherScatterMode.PROMISE_IN_BOUNDS)


N16 = 10016


def _edge_body(xl_hbm, xr_hbm, src_hbm, dst_hbm, att_hbm,
               nums_hbm, dens_hbm,
               sidx, didx, xlbuf, xrbuf, wbuf, attv,
               dtab, accn, sem):
    ci = lax.axis_index("c")
    si = lax.axis_index("s")
    wid = si * NCORE + ci

    z16 = jnp.zeros((16,), jnp.float32)
    zi16 = jnp.zeros((16,), jnp.int32)

    def zero_row(e, _):
        for k in range(C // 16):
            wbuf[e, pl.ds(16 * k, 16)] = z16
        return 0

    lax.fori_loop(0, B, zero_row, 0)

    def zero_den(i, _):
        dtab[pl.ds(pl.multiple_of(i * 16, 16), 16)] = z16
        return 0

    lax.fori_loop(0, N16 // 16, zero_den, 0)

    for j in range(_NCHUNK_CEIL):
        k = j * NSUB + si
        @pl.when(k < NCHUNK)
        def _():
            r0 = pl.multiple_of(k * B, 8)
            pltpu.sync_copy(wbuf, accn.at[pl.ds(r0, B)])

    pltpu.sync_copy(att_hbm, attv)
    attregs = [attv[pl.ds(16 * k, 16)] for k in range(C // 16)]
    lanes = lax.iota(jnp.int32, 16)
    lane0 = lanes == 0
    xor_idx = [lanes ^ sh for sh in (1, 2, 4, 8)]

    plsc.subcore_barrier()

    def batch_body(b, _):
        base = pl.multiple_of(wid * EPW + b * B, 8)
        pltpu.sync_copy(src_hbm.at[pl.ds(base, B)], sidx)
        pltpu.sync_copy(dst_hbm.at[pl.ds(base, B)], didx)
        pltpu.async_copy(xl_hbm.at[sidx], xlbuf, sem).wait()
        pltpu.async_copy(xr_hbm.at[didx], xrbuf, sem).wait()

        def group_body(g, _):
            dvec = didx[pl.ds(pl.multiple_of(g * 16, 16), 16)]
            for j in range(16):
                e = g * 16 + j
                xs = [xlbuf[e, pl.ds(16 * k, 16)] for k in range(C // 16)]
                part = z16
                for k in range(C // 16):
                    v = xs[k] + xrbuf[e, pl.ds(16 * k, 16)]
                    part = part + jnp.maximum(v, 0.2 * v) * attregs[k]
                for xi in xor_idx:
                    part = part + _lane_shuffle(part, xi)
                ex = jnp.exp(part)
                for k in range(C // 16):
                    wbuf[e, pl.ds(16 * k, 16)] = ex * xs[k]
                dst_e = dvec[j]
                base8 = pl.multiple_of((dst_e // 8) * 8, 8)
                sl = pl.ds(base8, 16)
                dtab[sl] = dtab[sl] + jnp.where(lanes == dst_e - base8,
                                                ex, 0.0)
            return 0

        lax.fori_loop(0, B // 16, group_body, 0)
        pltpu.sync_copy(wbuf, accn.at[didx], add=True)
        return 0

    lax.fori_loop(0, NB, batch_body, 0)

    plsc.subcore_barrier()

    pltpu.sync_copy(dtab, dens_hbm.at[ci, si, 0])
    for j in range(_NCHUNK_CEIL):
        k = j * NSUB + si
        @pl.when(k < NCHUNK)
        def _():
            r0 = pl.multiple_of(k * B, 8)
            pltpu.sync_copy(accn.at[pl.ds(r0, B)],
                            nums_hbm.at[ci, pl.ds(r0, B)])


_edge_pass = pl.kernel(
    _edge_body,
    out_type=(
        jax.ShapeDtypeStruct((NCORE, NPAD, C), jnp.float32),
        jax.ShapeDtypeStruct((NCORE, NSUB, 1, N16), jnp.float32),
    ),
    mesh=plsc.VectorSubcoreMesh(
        core_axis_name="c", subcore_axis_name="s",
        num_cores=NCORE, num_subcores=NSUB),
    scratch_types=[
        pltpu.VMEM((B,), jnp.int32),
        pltpu.VMEM((B,), jnp.int32),
        pltpu.VMEM((B, C), jnp.float32),
        pltpu.VMEM((B, C), jnp.float32),
        pltpu.VMEM((B, C), jnp.float32),
        pltpu.VMEM((C,), jnp.float32),
        pltpu.VMEM((N16,), jnp.float32),
        pltpu.VMEM_SHARED((NPAD, C), jnp.float32),
        pltpu.SemaphoreType.DMA,
    ],
)


RB = 1000
RC = 400


def _proj2_body(x_ref, wl_ref, bl_ref, wr_ref, br_ref, xl_ref, xr_ref):
    xv = x_ref[...]
    xl_ref[...] = jnp.dot(xv, wl_ref[...],
                          preferred_element_type=jnp.float32) + bl_ref[...]
    xr_ref[...] = jnp.dot(xv, wr_ref[...],
                          preferred_element_type=jnp.float32) + br_ref[...]


def _proj2(x, wl, bl, wr, br):
    g = N // RB
    return pl.pallas_call(
        _proj2_body,
        grid=(g,),
        in_specs=[
            pl.BlockSpec((RB, C), lambda i: (i, 0)),
            pl.BlockSpec((C, C), lambda i: (0, 0)),
            pl.BlockSpec((1, C), lambda i: (0, 0)),
            pl.BlockSpec((C, C), lambda i: (0, 0)),
            pl.BlockSpec((1, C), lambda i: (0, 0)),
        ],
        out_specs=[
            pl.BlockSpec((RB, C), lambda i: (i, 0)),
            pl.BlockSpec((RB, C), lambda i: (i, 0)),
        ],
        out_shape=[
            jax.ShapeDtypeStruct((N, C), jnp.float32),
            jax.ShapeDtypeStruct((N, C), jnp.float32),
        ],
    )(x, wl, bl, wr, br)


def _combine_body(nums_ref, dens_ref, b_ref, h_ref, st_ref):
    n = nums_ref[0] + nums_ref[1]
    d = dens_ref[...]
    den = jnp.sum(d, axis=1, keepdims=True) + 1e-16
    h = n / den + b_ref[...]
    h_ref[...] = h

    @pl.when(pl.program_id(0) == 0)
    def _():
        st_ref[...] = jnp.zeros_like(st_ref)

    sh = jnp.sum(h, axis=0, keepdims=True)
    sh2 = jnp.sum(h * h, axis=0, keepdims=True)
    st_ref[...] += jnp.concatenate(
        [sh, sh2, jnp.zeros((6, C), jnp.float32)], axis=0)


def _combine(nums, dens, b):
    g = N // RC
    return pl.pallas_call(
        _combine_body,
        grid=(g,),
        in_specs=[
            pl.BlockSpec((NCORE, RC, C), lambda i: (0, i, 0)),
            pl.BlockSpec((RC, NWORK), lambda i: (i, 0)),
            pl.BlockSpec((1, C), lambda i: (0, 0)),
        ],
        out_specs=[
            pl.BlockSpec((RC, C), lambda i: (i, 0)),
            pl.BlockSpec((8, C), lambda i: (0, 0)),
        ],
        out_shape=[
            jax.ShapeDtypeStruct((N, C), jnp.float32),
            jax.ShapeDtypeStruct((8, C), jnp.float32),
        ],
    )(nums, dens, b)


def _bn_from_stats(h, st):
    mu = st[0:1] * (1.0 / N)
    var = st[1:2] * (1.0 / N) - mu * mu
    inv = lax.rsqrt(var + 1e-5)
    return (h - mu) * inv


def _bnproj_body(h_ref, st_ref, g_ref, be_ref,
                 wl_ref, bl_ref, wr_ref, br_ref, xl_ref, xr_ref):
    hn = _bn_from_stats(h_ref[...], st_ref[...])
    h = jnp.maximum(g_ref[...] * hn + be_ref[...], 0.0)
    xl_ref[...] = jnp.dot(h, wl_ref[...],
                          preferred_element_type=jnp.float32) + bl_ref[...]
    xr_ref[...] = jnp.dot(h, wr_ref[...],
                          preferred_element_type=jnp.float32) + br_ref[...]


def _bnproj(h, st, gg, be, wl, bl, wr, br):
    g = N // RB
    return pl.pallas_call(
        _bnproj_body,
        grid=(g,),
        in_specs=[
            pl.BlockSpec((RB, C), lambda i: (i, 0)),
            pl.BlockSpec((8, C), lambda i: (0, 0)),
            pl.BlockSpec((1, C), lambda i: (0, 0)),
            pl.BlockSpec((1, C), lambda i: (0, 0)),
            pl.BlockSpec((C, C), lambda i: (0, 0)),
            pl.BlockSpec((1, C), lambda i: (0, 0)),
            pl.BlockSpec((C, C), lambda i: (0, 0)),
            pl.BlockSpec((1, C), lambda i: (0, 0)),
        ],
        out_specs=[
            pl.BlockSpec((RB, C), lambda i: (i, 0)),
            pl.BlockSpec((RB, C), lambda i: (i, 0)),
        ],
        out_shape=[
            jax.ShapeDtypeStruct((N, C), jnp.float32),
            jax.ShapeDtypeStruct((N, C), jnp.float32),
        ],
    )(h, st, gg, be, wl, bl, wr, br)


def _final_body(h_ref, st_ref, g_ref, be_ref,
                w1_ref, b1_ref, w2_ref, b2_ref, o_ref):
    hn = _bn_from_stats(h_ref[...], st_ref[...])
    h = jnp.maximum(g_ref[...] * hn + be_ref[...], 0.0)
    z = jnp.maximum(jnp.dot(h, w1_ref[...],
                            preferred_element_type=jnp.float32) + b1_ref[...],
                    0.0)
    o_ref[...] = jnp.dot(z, w2_ref[...],
                         preferred_element_type=jnp.float32) + b2_ref[...]


def _final(h, st, gg, be, w1, b1, w2, b2):
    g = N // RB
    return pl.pallas_call(
        _final_body,
        grid=(g,),
        in_specs=[
            pl.BlockSpec((RB, C), lambda i: (i, 0)),
            pl.BlockSpec((8, C), lambda i: (0, 0)),
            pl.BlockSpec((1, C), lambda i: (0, 0)),
            pl.BlockSpec((1, C), lambda i: (0, 0)),
            pl.BlockSpec((C, C), lambda i: (0, 0)),
            pl.BlockSpec((1, C), lambda i: (0, 0)),
            pl.BlockSpec((C, OUT), lambda i: (0, 0)),
            pl.BlockSpec((1, OUT), lambda i: (0, 0)),
        ],
        out_specs=pl.BlockSpec((RB, OUT), lambda i: (i, 0)),
        out_shape=jax.ShapeDtypeStruct((N, OUT), jnp.float32),
    )(h, st, gg, be, w1, b1, w2, b2)


def kernel(x, edge_index, Wl0, bl0, Wr0, br0, att0, b0, g0, be0,
           Wl1, bl1, Wr1, br1, att1, b1, g1, be1, W1, bv1, W2, bv2):
    src = edge_index[0]
    dst = edge_index[1]
    r = lambda v: v.reshape(1, -1)

    dt = lambda d: d.reshape(NWORK, N16).T

    xl0, xr0 = _proj2(x, Wl0, r(bl0), Wr0, r(br0))
    nums0, dens0 = _edge_pass(xl0, xr0, src, dst, att0.reshape(C))
    h0, st0 = _combine(nums0, dt(dens0), r(b0))
    xl1, xr1 = _bnproj(h0, st0, r(g0), r(be0), Wl1, r(bl1), Wr1, r(br1))
    nums1, dens1 = _edge_pass(xl1, xr1, src, dst, att1.reshape(C))
    h1, st1 = _combine(nums1, dt(dens1), r(b1))
    return _final(h1, st1, r(g1), r(be1), W1, r(bv1), W2, r(bv2))


# speedup vs baseline: 8.3584x; 1.1391x over previous
import functools

import jax
import jax.numpy as jnp
from jax import lax
from jax.experimental import pallas as pl
from jax.experimental.pallas import tpu as pltpu
from jax.experimental.pallas import tpu_sc as plsc

N = 10000
E = 320000
C = 128
OUT = 64
NPAD = 10000
NCORE = 2
NSUB = 16
NWORK = NCORE * NSUB
EPW = E // NWORK
B = 80
NB = EPW // B
SUB = 80
NSUB_B = 5
BSUP = SUB * NSUB_B
NSUP = EPW // BSUP
NCHUNK = NPAD // 80
_NCHUNK_CEIL = -(-NCHUNK // NSUB)


_GDN = lax.GatherDimensionNumbers(
    offset_dims=(), collapsed_slice_dims=(0,), start_index_map=(0,))


def _lane_shuffle(v, idx):
    return lax.gather(v, idx[:, None], _GDN, slice_sizes=(1,),
                      mode=lax.GatherScatterMode.PROMISE_IN_BOUNDS)


N16 = 10016
ZB = 80


def _edge_body(xl_hbm, xr_hbm, esd_hbm, att_hbm,
               nums_hbm, dens_hbm,
               e32, sidx, ds0, xlb, xrb, wb, attv, dtab, accn, sem):
    ci = lax.axis_index("c")
    si = lax.axis_index("s")

    z16 = jnp.zeros((16,), jnp.float32)

    def zero_row(e, _):
        for k in range(C // 16):
            wb[e, pl.ds(16 * k, 16)] = z16
        return 0

    lax.fori_loop(0, ZB, zero_row, 0)

    def zero_den(i, _):
        dtab[pl.ds(pl.multiple_of(i * 16, 16), 16)] = z16
        return 0

    lax.fori_loop(0, N16 // 16, zero_den, 0)

    for j in range(_NCHUNK_CEIL):
        k = j * NSUB + si
        @pl.when(k < NCHUNK)
        def _():
            r0 = pl.multiple_of(k * ZB, 8)
            pltpu.sync_copy(wb.at[pl.ds(0, ZB)], accn.at[pl.ds(r0, ZB)])

    pltpu.sync_copy(att_hbm, attv)
    attregs = [attv[pl.ds(16 * k, 16)] for k in range(C // 16)]
    lanes = lax.iota(jnp.int32, 16)
    xor_idx = [lanes ^ sh for sh in (1, 2, 4, 8)]

    plsc.subcore_barrier()

    def batch_body(b, _):
        pltpu.sync_copy(esd_hbm.at[ci, si, b], e32)
        for h in range(B // 16):
            sl16 = pl.ds(16 * h, 16)
            sidx[sl16] = e32[0, 0, sl16]
            ds0[sl16] = e32[1, 0, sl16]
        d1 = pltpu.async_copy(xl_hbm.at[sidx], xlb, sem)
        d2 = pltpu.async_copy(xr_hbm.at[ds0], xrb, sem)
        d1.wait()
        d2.wait()

        def group_body(g, _):
            goff = pl.multiple_of(g * 16, 16)
            dvec = ds0[pl.ds(goff, 16)]
            for jj in range(16):
                e = goff + jj
                xs = [xlb[e, pl.ds(16 * k, 16)] for k in range(C // 16)]
                part = z16
                for k in range(C // 16):
                    v = xs[k] + xrb[e, pl.ds(16 * k, 16)]
                    part = part + jnp.maximum(v, 0.2 * v) * attregs[k]
                for xi in xor_idx:
                    part = part + _lane_shuffle(part, xi)
                ex = jnp.exp(part)
                for k in range(C // 16):
                    wb[e, pl.ds(16 * k, 16)] = ex * xs[k]
                dst_e = dvec[jj]
                base8 = pl.multiple_of((dst_e // 8) * 8, 8)
                sl = pl.ds(base8, 16)
                dtab[sl] = dtab[sl] + jnp.where(lanes == dst_e - base8,
                                                ex, 0.0)
            return 0

        lax.fori_loop(0, B // 16, group_body, 0)
        pltpu.sync_copy(wb, accn.at[ds0], add=True)
        return 0

    lax.fori_loop(0, NB, batch_body, 0)

    plsc.subcore_barrier()

    pltpu.sync_copy(dtab, dens_hbm.at[ci, si, 0])
    for j in range(_NCHUNK_CEIL):
        k = j * NSUB + si
        @pl.when(k < NCHUNK)
        def _():
            r0 = pl.multiple_of(k * ZB, 8)
            pltpu.sync_copy(accn.at[pl.ds(r0, ZB)],
                            nums_hbm.at[ci, pl.ds(r0, ZB)])


_edge_pass = pl.kernel(
    _edge_body,
    out_type=(
        jax.ShapeDtypeStruct((NCORE, NPAD, C), jnp.float32),
        jax.ShapeDtypeStruct((NCORE, NSUB, 1, N16), jnp.float32),
    ),
    mesh=plsc.VectorSubcoreMesh(
        core_axis_name="c", subcore_axis_name="s",
        num_cores=NCORE, num_subcores=NSUB),
    scratch_types=[
        pltpu.VMEM((2, 1, B), jnp.int32),
        pltpu.VMEM((B,), jnp.int32),
        pltpu.VMEM((B,), jnp.int32),
        pltpu.VMEM((B, C), jnp.float32),
        pltpu.VMEM((B, C), jnp.float32),
        pltpu.VMEM((B, C), jnp.float32),
        pltpu.VMEM((C,), jnp.float32),
        pltpu.VMEM((N16,), jnp.float32),
        pltpu.VMEM_SHARED((NPAD, C), jnp.float32),
        pltpu.SemaphoreType.DMA,
    ],
)


RB = 1000
RC = 400


def _proj2_body(x_ref, wl_ref, bl_ref, wr_ref, br_ref, xl_ref, xr_ref):
    xv = x_ref[...]
    xl_ref[...] = jnp.dot(xv, wl_ref[...],
                          preferred_element_type=jnp.float32) + bl_ref[...]
    xr_ref[...] = jnp.dot(xv, wr_ref[...],
                          preferred_element_type=jnp.float32) + br_ref[...]


def _proj2(x, wl, bl, wr, br):
    g = N // RB
    return pl.pallas_call(
        _proj2_body,
        grid=(g,),
        in_specs=[
            pl.BlockSpec((RB, C), lambda i: (i, 0)),
            pl.BlockSpec((C, C), lambda i: (0, 0)),
            pl.BlockSpec((1, C), lambda i: (0, 0)),
            pl.BlockSpec((C, C), lambda i: (0, 0)),
            pl.BlockSpec((1, C), lambda i: (0, 0)),
        ],
        out_specs=[
            pl.BlockSpec((RB, C), lambda i: (i, 0)),
            pl.BlockSpec((RB, C), lambda i: (i, 0)),
        ],
        out_shape=[
            jax.ShapeDtypeStruct((N, C), jnp.float32),
            jax.ShapeDtypeStruct((N, C), jnp.float32),
        ],
    )(x, wl, bl, wr, br)


def _combine_body(nums_ref, dens_ref, b_ref, h_ref, st_ref):
    n = nums_ref[0] + nums_ref[1]
    d = dens_ref[...]
    den = jnp.sum(d, axis=1, keepdims=True) + 1e-16
    h = n / den + b_ref[...]
    h_ref[...] = h

    @pl.when(pl.program_id(0) == 0)
    def _():
        st_ref[...] = jnp.zeros_like(st_ref)

    sh = jnp.sum(h, axis=0, keepdims=True)
    sh2 = jnp.sum(h * h, axis=0, keepdims=True)
    st_ref[...] += jnp.concatenate(
        [sh, sh2, jnp.zeros((6, C), jnp.float32)], axis=0)


def _combine(nums, dens, b):
    g = N // RC
    return pl.pallas_call(
        _combine_body,
        grid=(g,),
        in_specs=[
            pl.BlockSpec((NCORE, RC, C), lambda i: (0, i, 0)),
            pl.BlockSpec((RC, NWORK), lambda i: (i, 0)),
            pl.BlockSpec((1, C), lambda i: (0, 0)),
        ],
        out_specs=[
            pl.BlockSpec((RC, C), lambda i: (i, 0)),
            pl.BlockSpec((8, C), lambda i: (0, 0)),
        ],
        out_shape=[
            jax.ShapeDtypeStruct((N, C), jnp.float32),
            jax.ShapeDtypeStruct((8, C), jnp.float32),
        ],
    )(nums, dens, b)


def _bn_from_stats(h, st):
    mu = st[0:1] * (1.0 / N)
    var = st[1:2] * (1.0 / N) - mu * mu
    inv = lax.rsqrt(var + 1e-5)
    return (h - mu) * inv


def _bnproj_body(h_ref, st_ref, g_ref, be_ref,
                 wl_ref, bl_ref, wr_ref, br_ref, xl_ref, xr_ref):
    hn = _bn_from_stats(h_ref[...], st_ref[...])
    h = jnp.maximum(g_ref[...] * hn + be_ref[...], 0.0)
    xl_ref[...] = jnp.dot(h, wl_ref[...],
                          preferred_element_type=jnp.float32) + bl_ref[...]
    xr_ref[...] = jnp.dot(h, wr_ref[...],
                          preferred_element_type=jnp.float32) + br_ref[...]


def _bnproj(h, st, gg, be, wl, bl, wr, br):
    g = N // RB
    return pl.pallas_call(
        _bnproj_body,
        grid=(g,),
        in_specs=[
            pl.BlockSpec((RB, C), lambda i: (i, 0)),
            pl.BlockSpec((8, C), lambda i: (0, 0)),
            pl.BlockSpec((1, C), lambda i: (0, 0)),
            pl.BlockSpec((1, C), lambda i: (0, 0)),
            pl.BlockSpec((C, C), lambda i: (0, 0)),
            pl.BlockSpec((1, C), lambda i: (0, 0)),
            pl.BlockSpec((C, C), lambda i: (0, 0)),
            pl.BlockSpec((1, C), lambda i: (0, 0)),
        ],
        out_specs=[
            pl.BlockSpec((RB, C), lambda i: (i, 0)),
            pl.BlockSpec((RB, C), lambda i: (i, 0)),
        ],
        out_shape=[
            jax.ShapeDtypeStruct((N, C), jnp.float32),
            jax.ShapeDtypeStruct((N, C), jnp.float32),
        ],
    )(h, st, gg, be, wl, bl, wr, br)


def _final_body(h_ref, st_ref, g_ref, be_ref,
                w1_ref, b1_ref, w2_ref, b2_ref, o_ref):
    hn = _bn_from_stats(h_ref[...], st_ref[...])
    h = jnp.maximum(g_ref[...] * hn + be_ref[...], 0.0)
    z = jnp.maximum(jnp.dot(h, w1_ref[...],
                            preferred_element_type=jnp.float32) + b1_ref[...],
                    0.0)
    o_ref[...] = jnp.dot(z, w2_ref[...],
                         preferred_element_type=jnp.float32) + b2_ref[...]


def _final(h, st, gg, be, w1, b1, w2, b2):
    g = N // RB
    return pl.pallas_call(
        _final_body,
        grid=(g,),
        in_specs=[
            pl.BlockSpec((RB, C), lambda i: (i, 0)),
            pl.BlockSpec((8, C), lambda i: (0, 0)),
            pl.BlockSpec((1, C), lambda i: (0, 0)),
            pl.BlockSpec((1, C), lambda i: (0, 0)),
            pl.BlockSpec((C, C), lambda i: (0, 0)),
            pl.BlockSpec((1, C), lambda i: (0, 0)),
            pl.BlockSpec((C, OUT), lambda i: (0, 0)),
            pl.BlockSpec((1, OUT), lambda i: (0, 0)),
        ],
        out_specs=pl.BlockSpec((RB, OUT), lambda i: (i, 0)),
        out_shape=jax.ShapeDtypeStruct((N, OUT), jnp.float32),
    )(h, st, gg, be, w1, b1, w2, b2)


def kernel(x, edge_index, Wl0, bl0, Wr0, br0, att0, b0, g0, be0,
           Wl1, bl1, Wr1, br1, att1, b1, g1, be1, W1, bv1, W2, bv2):
    src = edge_index[0].reshape(NWORK * NB, B)
    dst = edge_index[1].reshape(NWORK * NB, B)
    esd16 = jnp.stack([src, dst], axis=1).reshape(
        NSUB, NCORE, NB, 2, 1, B).swapaxes(0, 1)
    r = lambda v: v.reshape(1, -1)

    dt = lambda d: d.reshape(NWORK, N16).T

    xl0, xr0 = _proj2(x, Wl0, r(bl0), Wr0, r(br0))
    nums0, dens0 = _edge_pass(xl0, xr0, esd16, att0.reshape(C))
    h0, st0 = _combine(nums0, dt(dens0), r(b0))
    xl1, xr1 = _bnproj(h0, st0, r(g0), r(be0), Wl1, r(bl1), Wr1, r(br1))
    nums1, dens1 = _edge_pass(xl1, xr1, esd16, att1.reshape(C))
    h1, st1 = _combine(nums1, dt(dens1), r(b1))
    return _final(h1, st1, r(g1), r(be1), W1, r(bv1), W2, r(bv2))


# speedup vs baseline: 8.9447x; 1.0701x over previous
import functools

import jax
import jax.numpy as jnp
from jax import lax
from jax.experimental import pallas as pl
from jax.experimental.pallas import tpu as pltpu
from jax.experimental.pallas import tpu_sc as plsc

N = 10000
E = 320000
C = 128
OUT = 64
NPAD = 10000
NCORE = 2
NSUB = 16
NWORK = NCORE * NSUB
EPW = E // NWORK
B = 80
NB = EPW // B
SUB = 80
NSUB_B = 5
BSUP = SUB * NSUB_B
NSUP = EPW // BSUP
NCHUNK = NPAD // 80
_NCHUNK_CEIL = -(-NCHUNK // NSUB)


_GDN = lax.GatherDimensionNumbers(
    offset_dims=(), collapsed_slice_dims=(0,), start_index_map=(0,))


def _lane_shuffle(v, idx):
    return lax.gather(v, idx[:, None], _GDN, slice_sizes=(1,),
                      mode=lax.GatherScatterMode.PROMISE_IN_BOUNDS)


N16 = 10016
ZB = 80


def _edge_body(xl_hbm, xr_hbm, esd_hbm, att_hbm,
               nums_hbm, dens_hbm,
               e32, sidx, ds0, dscat, xlb, xrb, wb, attv, dtab, accn,
               sem, wsem):
    ci = lax.axis_index("c")
    si = lax.axis_index("s")

    z16 = jnp.zeros((16,), jnp.float32)

    def zero_row(e, _):
        for k in range(C // 16):
            wb[e, pl.ds(16 * k, 16)] = z16
        return 0

    lax.fori_loop(0, ZB, zero_row, 0)

    def zero_den(i, _):
        dtab[pl.ds(pl.multiple_of(i * 16, 16), 16)] = z16
        return 0

    lax.fori_loop(0, N16 // 16, zero_den, 0)

    for j in range(_NCHUNK_CEIL):
        k = j * NSUB + si
        @pl.when(k < NCHUNK)
        def _():
            r0 = pl.multiple_of(k * ZB, 8)
            pltpu.sync_copy(wb.at[pl.ds(0, ZB)], accn.at[pl.ds(r0, ZB)])

    pltpu.sync_copy(att_hbm, attv)
    attregs = [attv[pl.ds(16 * k, 16)] for k in range(C // 16)]
    lanes = lax.iota(jnp.int32, 16)
    xor_idx = [lanes ^ sh for sh in (1, 2, 4, 8)]

    plsc.subcore_barrier()

    def batch_body(b, _):
        pltpu.sync_copy(esd_hbm.at[ci, si, b], e32)
        for h in range(B // 16):
            sl16 = pl.ds(16 * h, 16)
            sidx[sl16] = e32[0, 0, sl16]
            ds0[sl16] = e32[1, 0, sl16]
        d1 = pltpu.async_copy(xl_hbm.at[sidx], xlb, sem)
        d2 = pltpu.async_copy(xr_hbm.at[ds0], xrb, sem)

        @pl.when(b > 0)
        def _():
            pltpu.make_async_copy(wb, accn.at[dscat], wsem).wait()

        d1.wait()
        d2.wait()

        def group_body(g, _):
            goff = pl.multiple_of(g * 16, 16)
            dvec = ds0[pl.ds(goff, 16)]
            for jj in range(16):
                e = goff + jj
                xs = [xlb[e, pl.ds(16 * k, 16)] for k in range(C // 16)]
                part = z16
                for k in range(C // 16):
                    v = xs[k] + xrb[e, pl.ds(16 * k, 16)]
                    part = part + jnp.maximum(v, 0.2 * v) * attregs[k]
                for xi in xor_idx:
                    part = part + _lane_shuffle(part, xi)
                ex = jnp.exp(part)
                for k in range(C // 16):
                    wb[e, pl.ds(16 * k, 16)] = ex * xs[k]
                dst_e = dvec[jj]
                base8 = pl.multiple_of((dst_e // 8) * 8, 8)
                sl = pl.ds(base8, 16)
                dtab[sl] = dtab[sl] + jnp.where(lanes == dst_e - base8,
                                                ex, 0.0)
            return 0

        lax.fori_loop(0, B // 16, group_body, 0)
        for h in range(B // 16):
            sl16 = pl.ds(16 * h, 16)
            dscat[sl16] = ds0[sl16]
        pltpu.async_copy(wb, accn.at[dscat], wsem, add=True)
        return 0

    lax.fori_loop(0, NB, batch_body, 0)
    pltpu.make_async_copy(wb, accn.at[dscat], wsem).wait()

    plsc.subcore_barrier()

    pltpu.sync_copy(dtab, dens_hbm.at[ci, si, 0])
    for j in range(_NCHUNK_CEIL):
        k = j * NSUB + si
        @pl.when(k < NCHUNK)
        def _():
            r0 = pl.multiple_of(k * ZB, 8)
            pltpu.sync_copy(accn.at[pl.ds(r0, ZB)],
                            nums_hbm.at[ci, pl.ds(r0, ZB)])


_edge_pass = pl.kernel(
    _edge_body,
    out_type=(
        jax.ShapeDtypeStruct((NCORE, NPAD, C), jnp.float32),
        jax.ShapeDtypeStruct((NCORE, NSUB, 1, N16), jnp.float32),
    ),
    mesh=plsc.VectorSubcoreMesh(
        core_axis_name="c", subcore_axis_name="s",
        num_cores=NCORE, num_subcores=NSUB),
    scratch_types=[
        pltpu.VMEM((2, 1, B), jnp.int32),
        pltpu.VMEM((B,), jnp.int32),
        pltpu.VMEM((B,), jnp.int32),
        pltpu.VMEM((B,), jnp.int32),
        pltpu.VMEM((B, C), jnp.float32),
        pltpu.VMEM((B, C), jnp.float32),
        pltpu.VMEM((B, C), jnp.float32),
        pltpu.VMEM((C,), jnp.float32),
        pltpu.VMEM((N16,), jnp.float32),
        pltpu.VMEM_SHARED((NPAD, C), jnp.float32),
        pltpu.SemaphoreType.DMA,
        pltpu.SemaphoreType.DMA,
    ],
)


RB = 1000
RC = 400


def _proj2_body(x_ref, wl_ref, bl_ref, wr_ref, br_ref, xl_ref, xr_ref):
    xv = x_ref[...]
    xl_ref[...] = jnp.dot(xv, wl_ref[...],
                          preferred_element_type=jnp.float32) + bl_ref[...]
    xr_ref[...] = jnp.dot(xv, wr_ref[...],
                          preferred_element_type=jnp.float32) + br_ref[...]


def _proj2(x, wl, bl, wr, br):
    g = N // RB
    return pl.pallas_call(
        _proj2_body,
        grid=(g,),
        in_specs=[
            pl.BlockSpec((RB, C), lambda i: (i, 0)),
            pl.BlockSpec((C, C), lambda i: (0, 0)),
            pl.BlockSpec((1, C), lambda i: (0, 0)),
            pl.BlockSpec((C, C), lambda i: (0, 0)),
            pl.BlockSpec((1, C), lambda i: (0, 0)),
        ],
        out_specs=[
            pl.BlockSpec((RB, C), lambda i: (i, 0)),
            pl.BlockSpec((RB, C), lambda i: (i, 0)),
        ],
        out_shape=[
            jax.ShapeDtypeStruct((N, C), jnp.float32),
            jax.ShapeDtypeStruct((N, C), jnp.float32),
        ],
    )(x, wl, bl, wr, br)


def _combine_body(nums_ref, dens_ref, b_ref, h_ref, st_ref):
    n = nums_ref[0] + nums_ref[1]
    d = dens_ref[...]
    den = jnp.sum(d, axis=1, keepdims=True) + 1e-16
    h = n / den + b_ref[...]
    h_ref[...] = h

    @pl.when(pl.program_id(0) == 0)
    def _():
        st_ref[...] = jnp.zeros_like(st_ref)

    sh = jnp.sum(h, axis=0, keepdims=True)
    sh2 = jnp.sum(h * h, axis=0, keepdims=True)
    st_ref[...] += jnp.concatenate(
        [sh, sh2, jnp.zeros((6, C), jnp.float32)], axis=0)


def _combine(nums, dens, b):
    g = N // RC
    return pl.pallas_call(
        _combine_body,
        grid=(g,),
        in_specs=[
            pl.BlockSpec((NCORE, RC, C), lambda i: (0, i, 0)),
            pl.BlockSpec((RC, NWORK), lambda i: (i, 0)),
            pl.BlockSpec((1, C), lambda i: (0, 0)),
        ],
        out_specs=[
            pl.BlockSpec((RC, C), lambda i: (i, 0)),
            pl.BlockSpec((8, C), lambda i: (0, 0)),
        ],
        out_shape=[
            jax.ShapeDtypeStruct((N, C), jnp.float32),
            jax.ShapeDtypeStruct((8, C), jnp.float32),
        ],
    )(nums, dens, b)


def _bn_from_stats(h, st):
    mu = st[0:1] * (1.0 / N)
    var = st[1:2] * (1.0 / N) - mu * mu
    inv = lax.rsqrt(var + 1e-5)
    return (h - mu) * inv


def _bnproj_body(h_ref, st_ref, g_ref, be_ref,
                 wl_ref, bl_ref, wr_ref, br_ref, xl_ref, xr_ref):
    hn = _bn_from_stats(h_ref[...], st_ref[...])
    h = jnp.maximum(g_ref[...] * hn + be_ref[...], 0.0)
    xl_ref[...] = jnp.dot(h, wl_ref[...],
                          preferred_element_type=jnp.float32) + bl_ref[...]
    xr_ref[...] = jnp.dot(h, wr_ref[...],
                          preferred_element_type=jnp.float32) + br_ref[...]


def _bnproj(h, st, gg, be, wl, bl, wr, br):
    g = N // RB
    return pl.pallas_call(
        _bnproj_body,
        grid=(g,),
        in_specs=[
            pl.BlockSpec((RB, C), lambda i: (i, 0)),
            pl.BlockSpec((8, C), lambda i: (0, 0)),
            pl.BlockSpec((1, C), lambda i: (0, 0)),
            pl.BlockSpec((1, C), lambda i: (0, 0)),
            pl.BlockSpec((C, C), lambda i: (0, 0)),
            pl.BlockSpec((1, C), lambda i: (0, 0)),
            pl.BlockSpec((C, C), lambda i: (0, 0)),
            pl.BlockSpec((1, C), lambda i: (0, 0)),
        ],
        out_specs=[
            pl.BlockSpec((RB, C), lambda i: (i, 0)),
            pl.BlockSpec((RB, C), lambda i: (i, 0)),
        ],
        out_shape=[
            jax.ShapeDtypeStruct((N, C), jnp.float32),
            jax.ShapeDtypeStruct((N, C), jnp.float32),
        ],
    )(h, st, gg, be, wl, bl, wr, br)


def _final_body(h_ref, st_ref, g_ref, be_ref,
                w1_ref, b1_ref, w2_ref, b2_ref, o_ref):
    hn = _bn_from_stats(h_ref[...], st_ref[...])
    h = jnp.maximum(g_ref[...] * hn + be_ref[...], 0.0)
    z = jnp.maximum(jnp.dot(h, w1_ref[...],
                            preferred_element_type=jnp.float32) + b1_ref[...],
                    0.0)
    o_ref[...] = jnp.dot(z, w2_ref[...],
                         preferred_element_type=jnp.float32) + b2_ref[...]


def _final(h, st, gg, be, w1, b1, w2, b2):
    g = N // RB
    return pl.pallas_call(
        _final_body,
        grid=(g,),
        in_specs=[
            pl.BlockSpec((RB, C), lambda i: (i, 0)),
            pl.BlockSpec((8, C), lambda i: (0, 0)),
            pl.BlockSpec((1, C), lambda i: (0, 0)),
            pl.BlockSpec((1, C), lambda i: (0, 0)),
            pl.BlockSpec((C, C), lambda i: (0, 0)),
            pl.BlockSpec((1, C), lambda i: (0, 0)),
            pl.BlockSpec((C, OUT), lambda i: (0, 0)),
            pl.BlockSpec((1, OUT), lambda i: (0, 0)),
        ],
        out_specs=pl.BlockSpec((RB, OUT), lambda i: (i, 0)),
        out_shape=jax.ShapeDtypeStruct((N, OUT), jnp.float32),
    )(h, st, gg, be, w1, b1, w2, b2)


def kernel(x, edge_index, Wl0, bl0, Wr0, br0, att0, b0, g0, be0,
           Wl1, bl1, Wr1, br1, att1, b1, g1, be1, W1, bv1, W2, bv2):
    src = edge_index[0].reshape(NWORK * NB, B)
    dst = edge_index[1].reshape(NWORK * NB, B)
    esd16 = jnp.stack([src, dst], axis=1).reshape(
        NSUB, NCORE, NB, 2, 1, B).swapaxes(0, 1)
    r = lambda v: v.reshape(1, -1)

    dt = lambda d: d.reshape(NWORK, N16).T

    xl0, xr0 = _proj2(x, Wl0, r(bl0), Wr0, r(br0))
    nums0, dens0 = _edge_pass(xl0, xr0, esd16, att0.reshape(C))
    h0, st0 = _combine(nums0, dt(dens0), r(b0))
    xl1, xr1 = _bnproj(h0, st0, r(g0), r(be0), Wl1, r(bl1), Wr1, r(br1))
    nums1, dens1 = _edge_pass(xl1, xr1, esd16, att1.reshape(C))
    h1, st1 = _combine(nums1, dt(dens1), r(b1))
    return _final(h1, st1, r(g1), r(be1), W1, r(bv1), W2, r(bv2))


# speedup vs baseline: 9.6784x; 1.0820x over previous
import functools

import jax
import jax.numpy as jnp
from jax import lax
from jax.experimental import pallas as pl
from jax.experimental.pallas import tpu as pltpu
from jax.experimental.pallas import tpu_sc as plsc

N = 10000
E = 320000
C = 128
OUT = 64
NPAD = 10000
NCORE = 2
NSUB = 16
NWORK = NCORE * NSUB
EPW = E // NWORK
B = 80
NB = EPW // B
SUB = 80
NSUB_B = 5
BSUP = SUB * NSUB_B
NSUP = EPW // BSUP
NCHUNK = NPAD // 80
_NCHUNK_CEIL = -(-NCHUNK // NSUB)


_GDN = lax.GatherDimensionNumbers(
    offset_dims=(), collapsed_slice_dims=(0,), start_index_map=(0,))


def _lane_shuffle(v, idx):
    return lax.gather(v, idx[:, None], _GDN, slice_sizes=(1,),
                      mode=lax.GatherScatterMode.PROMISE_IN_BOUNDS)


N16 = 10016
ZB = 80


def _edge_body(xl_hbm, xr_hbm, esd_hbm, att_hbm,
               nums_hbm, dens_hbm,
               e32, sidx, ds0, dscat, xlb, xrb, wb, attv, dtab, accn,
               sem, wsem, isem):
    ci = lax.axis_index("c")
    si = lax.axis_index("s")

    z16 = jnp.zeros((16,), jnp.float32)

    def zero_row(e, _):
        for k in range(C // 16):
            wb[e, pl.ds(16 * k, 16)] = z16
        return 0

    lax.fori_loop(0, ZB, zero_row, 0)

    def zero_den(i, _):
        dtab[pl.ds(pl.multiple_of(i * 16, 16), 16)] = z16
        return 0

    lax.fori_loop(0, N16 // 16, zero_den, 0)

    for j in range(_NCHUNK_CEIL):
        k = j * NSUB + si
        @pl.when(k < NCHUNK)
        def _():
            r0 = pl.multiple_of(k * ZB, 8)
            pltpu.sync_copy(wb.at[pl.ds(0, ZB)], accn.at[pl.ds(r0, ZB)])

    pltpu.sync_copy(att_hbm, attv)
    attregs = [attv[pl.ds(16 * k, 16)] for k in range(C // 16)]
    lanes = lax.iota(jnp.int32, 16)
    xor_idx = [lanes ^ sh for sh in (1, 2, 4, 8)]

    plsc.subcore_barrier()

    def batch_body(b, _):
        par = lax.rem(b, 2)
        pltpu.make_async_copy(esd_hbm.at[ci, si, b], e32.at[par],
                              isem).wait()
        for h in range(B // 16):
            sl16 = pl.ds(16 * h, 16)
            sidx[sl16] = e32[par, 0, 0, sl16]
            ds0[sl16] = e32[par, 1, 0, sl16]
        d1 = pltpu.async_copy(xl_hbm.at[sidx], xlb, sem)
        d2 = pltpu.async_copy(xr_hbm.at[ds0], xrb, sem)
        pltpu.async_copy(esd_hbm.at[ci, si, jnp.minimum(b + 1, NB - 1)],
                         e32.at[1 - par], isem)

        @pl.when(b > 0)
        def _():
            pltpu.make_async_copy(wb, accn.at[dscat], wsem).wait()

        d1.wait()
        d2.wait()

        def group_body(g, _):
            goff = pl.multiple_of(g * 16, 16)
            dvec = ds0[pl.ds(goff, 16)]
            for jj in range(16):
                e = goff + jj
                xs = [xlb[e, pl.ds(16 * k, 16)] for k in range(C // 16)]
                part = z16
                for k in range(C // 16):
                    v = xs[k] + xrb[e, pl.ds(16 * k, 16)]
                    part = part + jnp.maximum(v, 0.2 * v) * attregs[k]
                for xi in xor_idx:
                    part = part + _lane_shuffle(part, xi)
                ex = jnp.exp(part)
                for k in range(C // 16):
                    wb[e, pl.ds(16 * k, 16)] = ex * xs[k]
                dst_e = dvec[jj]
                base8 = pl.multiple_of((dst_e // 8) * 8, 8)
                sl = pl.ds(base8, 16)
                dtab[sl] = dtab[sl] + jnp.where(lanes == dst_e - base8,
                                                ex, 0.0)
            return 0

        lax.fori_loop(0, B // 16, group_body, 0)
        for h in range(B // 16):
            sl16 = pl.ds(16 * h, 16)
            dscat[sl16] = ds0[sl16]
        pltpu.async_copy(wb, accn.at[dscat], wsem, add=True)
        return 0

    pltpu.async_copy(esd_hbm.at[ci, si, 0], e32.at[0], isem)
    lax.fori_loop(0, NB, batch_body, 0)
    pltpu.make_async_copy(wb, accn.at[dscat], wsem).wait()
    pltpu.make_async_copy(esd_hbm.at[ci, si, 0], e32.at[lax.rem(NB, 2)],
                          isem).wait()

    plsc.subcore_barrier()

    pltpu.sync_copy(dtab, dens_hbm.at[ci, si, 0])
    for j in range(_NCHUNK_CEIL):
        k = j * NSUB + si
        @pl.when(k < NCHUNK)
        def _():
            r0 = pl.multiple_of(k * ZB, 8)
            pltpu.sync_copy(accn.at[pl.ds(r0, ZB)],
                            nums_hbm.at[ci, pl.ds(r0, ZB)])


_edge_pass = pl.kernel(
    _edge_body,
    out_type=(
        jax.ShapeDtypeStruct((NCORE, NPAD, C), jnp.float32),
        jax.ShapeDtypeStruct((NCORE, NSUB, 1, N16), jnp.float32),
    ),
    mesh=plsc.VectorSubcoreMesh(
        core_axis_name="c", subcore_axis_name="s",
        num_cores=NCORE, num_subcores=NSUB),
    scratch_types=[
        pltpu.VMEM((2, 2, 1, B), jnp.int32),
        pltpu.VMEM((B,), jnp.int32),
        pltpu.VMEM((B,), jnp.int32),
        pltpu.VMEM((B,), jnp.int32),
        pltpu.VMEM((B, C), jnp.float32),
        pltpu.VMEM((B, C), jnp.float32),
        pltpu.VMEM((B, C), jnp.float32),
        pltpu.VMEM((C,), jnp.float32),
        pltpu.VMEM((N16,), jnp.float32),
        pltpu.VMEM_SHARED((NPAD, C), jnp.float32),
        pltpu.SemaphoreType.DMA,
        pltpu.SemaphoreType.DMA,
        pltpu.SemaphoreType.DMA,
    ],
)


RB = 1000
RC = 400


def _proj2_body(x_ref, wl_ref, bl_ref, wr_ref, br_ref, xl_ref, xr_ref):
    xv = x_ref[...]
    xl_ref[...] = jnp.dot(xv, wl_ref[...],
                          preferred_element_type=jnp.float32) + bl_ref[...]
    xr_ref[...] = jnp.dot(xv, wr_ref[...],
                          preferred_element_type=jnp.float32) + br_ref[...]


def _proj2(x, wl, bl, wr, br):
    g = N // RB
    return pl.pallas_call(
        _proj2_body,
        grid=(g,),
        in_specs=[
            pl.BlockSpec((RB, C), lambda i: (i, 0)),
            pl.BlockSpec((C, C), lambda i: (0, 0)),
            pl.BlockSpec((1, C), lambda i: (0, 0)),
            pl.BlockSpec((C, C), lambda i: (0, 0)),
            pl.BlockSpec((1, C), lambda i: (0, 0)),
        ],
        out_specs=[
            pl.BlockSpec((RB, C), lambda i: (i, 0)),
            pl.BlockSpec((RB, C), lambda i: (i, 0)),
        ],
        out_shape=[
            jax.ShapeDtypeStruct((N, C), jnp.float32),
            jax.ShapeDtypeStruct((N, C), jnp.float32),
        ],
    )(x, wl, bl, wr, br)


def _combine_body(nums_ref, dens_ref, b_ref, h_ref, st_ref):
    n = nums_ref[0] + nums_ref[1]
    d = dens_ref[...]
    den = jnp.sum(d, axis=1, keepdims=True) + 1e-16
    h = n / den + b_ref[...]
    h_ref[...] = h

    @pl.when(pl.program_id(0) == 0)
    def _():
        st_ref[...] = jnp.zeros_like(st_ref)

    sh = jnp.sum(h, axis=0, keepdims=True)
    sh2 = jnp.sum(h * h, axis=0, keepdims=True)
    st_ref[...] += jnp.concatenate(
        [sh, sh2, jnp.zeros((6, C), jnp.float32)], axis=0)


def _combine(nums, dens, b):
    g = N // RC
    return pl.pallas_call(
        _combine_body,
        grid=(g,),
        in_specs=[
            pl.BlockSpec((NCORE, RC, C), lambda i: (0, i, 0)),
            pl.BlockSpec((RC, NWORK), lambda i: (i, 0)),
            pl.BlockSpec((1, C), lambda i: (0, 0)),
        ],
        out_specs=[
            pl.BlockSpec((RC, C), lambda i: (i, 0)),
            pl.BlockSpec((8, C), lambda i: (0, 0)),
        ],
        out_shape=[
            jax.ShapeDtypeStruct((N, C), jnp.float32),
            jax.ShapeDtypeStruct((8, C), jnp.float32),
        ],
    )(nums, dens, b)


def _bn_from_stats(h, st):
    mu = st[0:1] * (1.0 / N)
    var = st[1:2] * (1.0 / N) - mu * mu
    inv = lax.rsqrt(var + 1e-5)
    return (h - mu) * inv


def _bnproj_body(h_ref, st_ref, g_ref, be_ref,
                 wl_ref, bl_ref, wr_ref, br_ref, xl_ref, xr_ref):
    hn = _bn_from_stats(h_ref[...], st_ref[...])
    h = jnp.maximum(g_ref[...] * hn + be_ref[...], 0.0)
    xl_ref[...] = jnp.dot(h, wl_ref[...],
                          preferred_element_type=jnp.float32) + bl_ref[...]
    xr_ref[...] = jnp.dot(h, wr_ref[...],
                          preferred_element_type=jnp.float32) + br_ref[...]


def _bnproj(h, st, gg, be, wl, bl, wr, br):
    g = N // RB
    return pl.pallas_call(
        _bnproj_body,
        grid=(g,),
        in_specs=[
            pl.BlockSpec((RB, C), lambda i: (i, 0)),
            pl.BlockSpec((8, C), lambda i: (0, 0)),
            pl.BlockSpec((1, C), lambda i: (0, 0)),
            pl.BlockSpec((1, C), lambda i: (0, 0)),
            pl.BlockSpec((C, C), lambda i: (0, 0)),
            pl.BlockSpec((1, C), lambda i: (0, 0)),
            pl.BlockSpec((C, C), lambda i: (0, 0)),
            pl.BlockSpec((1, C), lambda i: (0, 0)),
        ],
        out_specs=[
            pl.BlockSpec((RB, C), lambda i: (i, 0)),
            pl.BlockSpec((RB, C), lambda i: (i, 0)),
        ],
        out_shape=[
            jax.ShapeDtypeStruct((N, C), jnp.float32),
            jax.ShapeDtypeStruct((N, C), jnp.float32),
        ],
    )(h, st, gg, be, wl, bl, wr, br)


def _final_body(h_ref, st_ref, g_ref, be_ref,
                w1_ref, b1_ref, w2_ref, b2_ref, o_ref):
    hn = _bn_from_stats(h_ref[...], st_ref[...])
    h = jnp.maximum(g_ref[...] * hn + be_ref[...], 0.0)
    z = jnp.maximum(jnp.dot(h, w1_ref[...],
                            preferred_element_type=jnp.float32) + b1_ref[...],
                    0.0)
    o_ref[...] = jnp.dot(z, w2_ref[...],
                         preferred_element_type=jnp.float32) + b2_ref[...]


def _final(h, st, gg, be, w1, b1, w2, b2):
    g = N // RB
    return pl.pallas_call(
        _final_body,
        grid=(g,),
        in_specs=[
            pl.BlockSpec((RB, C), lambda i: (i, 0)),
            pl.BlockSpec((8, C), lambda i: (0, 0)),
            pl.BlockSpec((1, C), lambda i: (0, 0)),
            pl.BlockSpec((1, C), lambda i: (0, 0)),
            pl.BlockSpec((C, C), lambda i: (0, 0)),
            pl.BlockSpec((1, C), lambda i: (0, 0)),
            pl.BlockSpec((C, OUT), lambda i: (0, 0)),
            pl.BlockSpec((1, OUT), lambda i: (0, 0)),
        ],
        out_specs=pl.BlockSpec((RB, OUT), lambda i: (i, 0)),
        out_shape=jax.ShapeDtypeStruct((N, OUT), jnp.float32),
    )(h, st, gg, be, w1, b1, w2, b2)


def kernel(x, edge_index, Wl0, bl0, Wr0, br0, att0, b0, g0, be0,
           Wl1, bl1, Wr1, br1, att1, b1, g1, be1, W1, bv1, W2, bv2):
    src = edge_index[0].reshape(NWORK * NB, B)
    dst = edge_index[1].reshape(NWORK * NB, B)
    esd16 = jnp.stack([src, dst], axis=1).reshape(
        NSUB, NCORE, NB, 2, 1, B).swapaxes(0, 1)
    r = lambda v: v.reshape(1, -1)

    dt = lambda d: d.reshape(NWORK, N16).T

    xl0, xr0 = _proj2(x, Wl0, r(bl0), Wr0, r(br0))
    nums0, dens0 = _edge_pass(xl0, xr0, esd16, att0.reshape(C))
    h0, st0 = _combine(nums0, dt(dens0), r(b0))
    xl1, xr1 = _bnproj(h0, st0, r(g0), r(be0), Wl1, r(bl1), Wr1, r(br1))
    nums1, dens1 = _edge_pass(xl1, xr1, esd16, att1.reshape(C))
    h1, st1 = _combine(nums1, dt(dens1), r(b1))
    return _final(h1, st1, r(g1), r(be1), W1, r(bv1), W2, r(bv2))


# speedup vs baseline: 9.6985x; 1.0021x over previous
import jax
import jax.numpy as jnp
from jax import lax
from jax.experimental import pallas as pl
from jax.experimental.pallas import tpu as pltpu
from jax.experimental.pallas import tpu_sc as plsc

N = 10000
E = 320000
C = 128
OUT = 64
NPAD = 10000
NCORE = 2
NSUB = 16
NWORK = NCORE * NSUB
EPW = E // NWORK
B = 80
NB = EPW // B
NCHUNK = NPAD // 80
_NCHUNK_CEIL = -(-NCHUNK // NSUB)


_GDN = lax.GatherDimensionNumbers(
    offset_dims=(), collapsed_slice_dims=(0,), start_index_map=(0,))


def _lane_shuffle(v, idx):
    return lax.gather(v, idx[:, None], _GDN, slice_sizes=(1,),
                      mode=lax.GatherScatterMode.PROMISE_IN_BOUNDS)


N16 = 10016
ZB = 80


def _edge_body(xl_hbm, xr_hbm, esd_hbm, att_hbm,
               nums_hbm, dens_hbm,
               e32, sidx, ds0, dscat, xlb, xrb, wb, attv, dtab, accn,
               sem, wsem, isem):
    ci = lax.axis_index("c")
    si = lax.axis_index("s")

    z16 = jnp.zeros((16,), jnp.float32)

    def zero_row(e, _):
        for k in range(C // 16):
            wb[e, pl.ds(16 * k, 16)] = z16
        return 0

    lax.fori_loop(0, ZB, zero_row, 0)

    def zero_den(i, _):
        dtab[pl.ds(pl.multiple_of(i * 16, 16), 16)] = z16
        return 0

    lax.fori_loop(0, N16 // 16, zero_den, 0)

    for j in range(_NCHUNK_CEIL):
        k = j * NSUB + si
        @pl.when(k < NCHUNK)
        def _():
            r0 = pl.multiple_of(k * ZB, 8)
            pltpu.sync_copy(wb.at[pl.ds(0, ZB)], accn.at[pl.ds(r0, ZB)])

    pltpu.sync_copy(att_hbm, attv)
    attregs = [attv[pl.ds(16 * k, 16)] for k in range(C // 16)]
    lanes = lax.iota(jnp.int32, 16)
    xor_idx = [lanes ^ sh for sh in (1, 2, 4, 8)]

    plsc.subcore_barrier()

    def batch_body(b, _):
        par = lax.rem(b, 2)
        pltpu.make_async_copy(esd_hbm.at[ci, si, b], e32.at[par],
                              isem).wait()
        for h in range(B // 16):
            sl16 = pl.ds(16 * h, 16)
            sidx[sl16] = e32[par, 0, 0, sl16]
            ds0[sl16] = e32[par, 1, 0, sl16]
        d1 = pltpu.async_copy(xl_hbm.at[sidx], xlb, sem)
        d2 = pltpu.async_copy(xr_hbm.at[ds0], xrb, sem)
        pltpu.async_copy(esd_hbm.at[ci, si, jnp.minimum(b + 1, NB - 1)],
                         e32.at[1 - par], isem)

        @pl.when(b > 0)
        def _():
            pltpu.make_async_copy(wb, accn.at[dscat], wsem).wait()

        d1.wait()
        d2.wait()

        def group_body(g, _):
            goff = pl.multiple_of(g * 16, 16)
            dvec = ds0[pl.ds(goff, 16)]
            for jj in range(16):
                e = goff + jj
                xs = [xlb[e, pl.ds(16 * k, 16)] for k in range(C // 16)]
                part = z16
                for k in range(C // 16):
                    v = xs[k] + xrb[e, pl.ds(16 * k, 16)]
                    part = part + jnp.maximum(v, 0.2 * v) * attregs[k]
                for xi in xor_idx:
                    part = part + _lane_shuffle(part, xi)
                ex = jnp.exp(part)
                for k in range(C // 16):
                    wb[e, pl.ds(16 * k, 16)] = ex * xs[k]
                dst_e = dvec[jj]
                base8 = pl.multiple_of((dst_e // 8) * 8, 8)
                sl = pl.ds(base8, 16)
                dtab[sl] = dtab[sl] + jnp.where(lanes == dst_e - base8,
                                                ex, 0.0)
            return 0

        lax.fori_loop(0, B // 16, group_body, 0)
        for h in range(B // 16):
            sl16 = pl.ds(16 * h, 16)
            dscat[sl16] = ds0[sl16]
        pltpu.async_copy(wb, accn.at[dscat], wsem, add=True)
        return 0

    pltpu.async_copy(esd_hbm.at[ci, si, 0], e32.at[0], isem)
    lax.fori_loop(0, NB, batch_body, 0)
    pltpu.make_async_copy(wb, accn.at[dscat], wsem).wait()
    pltpu.make_async_copy(esd_hbm.at[ci, si, 0], e32.at[lax.rem(NB, 2)],
                          isem).wait()

    plsc.subcore_barrier()

    pltpu.sync_copy(dtab, dens_hbm.at[ci, si, 0])
    for j in range(_NCHUNK_CEIL):
        k = j * NSUB + si
        @pl.when(k < NCHUNK)
        def _():
            r0 = pl.multiple_of(k * ZB, 8)
            pltpu.sync_copy(accn.at[pl.ds(r0, ZB)],
                            nums_hbm.at[ci, pl.ds(r0, ZB)])


_edge_pass = pl.kernel(
    _edge_body,
    out_type=(
        jax.ShapeDtypeStruct((NCORE, NPAD, C), jnp.float32),
        jax.ShapeDtypeStruct((NCORE, NSUB, 1, N16), jnp.float32),
    ),
    mesh=plsc.VectorSubcoreMesh(
        core_axis_name="c", subcore_axis_name="s",
        num_cores=NCORE, num_subcores=NSUB),
    scratch_types=[
        pltpu.VMEM((2, 2, 1, B), jnp.int32),
        pltpu.VMEM((B,), jnp.int32),
        pltpu.VMEM((B,), jnp.int32),
        pltpu.VMEM((B,), jnp.int32),
        pltpu.VMEM((B, C), jnp.float32),
        pltpu.VMEM((B, C), jnp.float32),
        pltpu.VMEM((B, C), jnp.float32),
        pltpu.VMEM((C,), jnp.float32),
        pltpu.VMEM((N16,), jnp.float32),
        pltpu.VMEM_SHARED((NPAD, C), jnp.float32),
        pltpu.SemaphoreType.DMA,
        pltpu.SemaphoreType.DMA,
        pltpu.SemaphoreType.DMA,
    ],
)


RB = 1000
RC = 400


def _proj2_body(x_ref, wl_ref, bl_ref, wr_ref, br_ref, xl_ref, xr_ref):
    xv = x_ref[...]
    xl_ref[...] = jnp.dot(xv, wl_ref[...],
                          preferred_element_type=jnp.float32) + bl_ref[...]
    xr_ref[...] = jnp.dot(xv, wr_ref[...],
                          preferred_element_type=jnp.float32) + br_ref[...]


def _proj2(x, wl, bl, wr, br):
    g = N // RB
    return pl.pallas_call(
        _proj2_body,
        grid=(g,),
        in_specs=[
            pl.BlockSpec((RB, C), lambda i: (i, 0)),
            pl.BlockSpec((C, C), lambda i: (0, 0)),
            pl.BlockSpec((1, C), lambda i: (0, 0)),
            pl.BlockSpec((C, C), lambda i: (0, 0)),
            pl.BlockSpec((1, C), lambda i: (0, 0)),
        ],
        out_specs=[
            pl.BlockSpec((RB, C), lambda i: (i, 0)),
            pl.BlockSpec((RB, C), lambda i: (i, 0)),
        ],
        out_shape=[
            jax.ShapeDtypeStruct((N, C), jnp.float32),
            jax.ShapeDtypeStruct((N, C), jnp.float32),
        ],
    )(x, wl, bl, wr, br)


def _combine_body(nums_ref, dens_ref, b_ref, h_ref, st_ref):
    n = nums_ref[0] + nums_ref[1]
    d = dens_ref[...]
    den = jnp.sum(d, axis=1, keepdims=True) + 1e-16
    h = n / den + b_ref[...]
    h_ref[...] = h

    @pl.when(pl.program_id(0) == 0)
    def _():
        st_ref[...] = jnp.zeros_like(st_ref)

    sh = jnp.sum(h, axis=0, keepdims=True)
    sh2 = jnp.sum(h * h, axis=0, keepdims=True)
    st_ref[...] += jnp.concatenate(
        [sh, sh2, jnp.zeros((6, C), jnp.float32)], axis=0)


def _combine(nums, dens, b):
    g = N // RC
    return pl.pallas_call(
        _combine_body,
        grid=(g,),
        in_specs=[
            pl.BlockSpec((NCORE, RC, C), lambda i: (0, i, 0)),
            pl.BlockSpec((RC, NWORK), lambda i: (i, 0)),
            pl.BlockSpec((1, C), lambda i: (0, 0)),
        ],
        out_specs=[
            pl.BlockSpec((RC, C), lambda i: (i, 0)),
            pl.BlockSpec((8, C), lambda i: (0, 0)),
        ],
        out_shape=[
            jax.ShapeDtypeStruct((N, C), jnp.float32),
            jax.ShapeDtypeStruct((8, C), jnp.float32),
        ],
    )(nums, dens, b)


def _bn_from_stats(h, st):
    mu = st[0:1] * (1.0 / N)
    var = st[1:2] * (1.0 / N) - mu * mu
    inv = lax.rsqrt(var + 1e-5)
    return (h - mu) * inv


def _bnproj_body(h_ref, st_ref, g_ref, be_ref,
                 wl_ref, bl_ref, wr_ref, br_ref, xl_ref, xr_ref):
    hn = _bn_from_stats(h_ref[...], st_ref[...])
    h = jnp.maximum(g_ref[...] * hn + be_ref[...], 0.0)
    xl_ref[...] = jnp.dot(h, wl_ref[...],
                          preferred_element_type=jnp.float32) + bl_ref[...]
    xr_ref[...] = jnp.dot(h, wr_ref[...],
                          preferred_element_type=jnp.float32) + br_ref[...]


def _bnproj(h, st, gg, be, wl, bl, wr, br):
    g = N // RB
    return pl.pallas_call(
        _bnproj_body,
        grid=(g,),
        in_specs=[
            pl.BlockSpec((RB, C), lambda i: (i, 0)),
            pl.BlockSpec((8, C), lambda i: (0, 0)),
            pl.BlockSpec((1, C), lambda i: (0, 0)),
            pl.BlockSpec((1, C), lambda i: (0, 0)),
            pl.BlockSpec((C, C), lambda i: (0, 0)),
            pl.BlockSpec((1, C), lambda i: (0, 0)),
            pl.BlockSpec((C, C), lambda i: (0, 0)),
            pl.BlockSpec((1, C), lambda i: (0, 0)),
        ],
        out_specs=[
            pl.BlockSpec((RB, C), lambda i: (i, 0)),
            pl.BlockSpec((RB, C), lambda i: (i, 0)),
        ],
        out_shape=[
            jax.ShapeDtypeStruct((N, C), jnp.float32),
            jax.ShapeDtypeStruct((N, C), jnp.float32),
        ],
    )(h, st, gg, be, wl, bl, wr, br)


def _final_body(h_ref, st_ref, g_ref, be_ref,
                w1_ref, b1_ref, w2_ref, b2_ref, o_ref):
    hn = _bn_from_stats(h_ref[...], st_ref[...])
    h = jnp.maximum(g_ref[...] * hn + be_ref[...], 0.0)
    z = jnp.maximum(jnp.dot(h, w1_ref[...],
                            preferred_element_type=jnp.float32) + b1_ref[...],
                    0.0)
    o_ref[...] = jnp.dot(z, w2_ref[...],
                         preferred_element_type=jnp.float32) + b2_ref[...]


def _final(h, st, gg, be, w1, b1, w2, b2):
    g = N // RB
    return pl.pallas_call(
        _final_body,
        grid=(g,),
        in_specs=[
            pl.BlockSpec((RB, C), lambda i: (i, 0)),
            pl.BlockSpec((8, C), lambda i: (0, 0)),
            pl.BlockSpec((1, C), lambda i: (0, 0)),
            pl.BlockSpec((1, C), lambda i: (0, 0)),
            pl.BlockSpec((C, C), lambda i: (0, 0)),
            pl.BlockSpec((1, C), lambda i: (0, 0)),
            pl.BlockSpec((C, OUT), lambda i: (0, 0)),
            pl.BlockSpec((1, OUT), lambda i: (0, 0)),
        ],
        out_specs=pl.BlockSpec((RB, OUT), lambda i: (i, 0)),
        out_shape=jax.ShapeDtypeStruct((N, OUT), jnp.float32),
    )(h, st, gg, be, w1, b1, w2, b2)


def kernel(x, edge_index, Wl0, bl0, Wr0, br0, att0, b0, g0, be0,
           Wl1, bl1, Wr1, br1, att1, b1, g1, be1, W1, bv1, W2, bv2):
    src = edge_index[0].reshape(NWORK * NB, B)
    dst = edge_index[1].reshape(NWORK * NB, B)
    esd16 = jnp.stack([src, dst], axis=1).reshape(
        NSUB, NCORE, NB, 2, 1, B).swapaxes(0, 1)
    r = lambda v: v.reshape(1, -1)

    dt = lambda d: d.reshape(NWORK, N16).T

    xl0, xr0 = _proj2(x, Wl0, r(bl0), Wr0, r(br0))
    nums0, dens0 = _edge_pass(xl0, xr0, esd16, att0.reshape(C))
    h0, st0 = _combine(nums0, dt(dens0), r(b0))
    xl1, xr1 = _bnproj(h0, st0, r(g0), r(be0), Wl1, r(bl1), Wr1, r(br1))
    nums1, dens1 = _edge_pass(xl1, xr1, esd16, att1.reshape(C))
    h1, st1 = _combine(nums1, dt(dens1), r(b1))
    return _final(h1, st1, r(g1), r(be1), W1, r(bv1), W2, r(bv2))
